# baseline (speedup 1.0000x reference)
import jax
import jax.numpy as jnp
from jax.experimental import pallas as pl
from jax.experimental.pallas import tpu as pltpu

_NCHUNK = 16


def _body(x_hbm, a_hbm, c_ref, w1_ref, b1_ref, wbig_ref, cb_ref, o_ref,
          a_vmem, x_vmem, xall, abf, d_vmem, sem_x, sem_a):
    B, N, D = x_hbm.shape
    ch = N // _NCHUNK

    cps_x = [
        pltpu.make_async_copy(x_hbm.at[b], x_vmem.at[b], sem_x.at[b])
        for b in range(B)
    ]
    cps_a = [
        pltpu.make_async_copy(
            a_hbm.at[pl.ds(i * ch, ch), :],
            a_vmem.at[pl.ds(i * ch, ch), :],
            sem_a.at[i],
        )
        for i in range(_NCHUNK)
    ]
    for b in range(B):
        cps_x[b].start()
        cps_a[b].start()
    for i in range(B, _NCHUNK):
        cps_a[i].start()

    for b in range(B):
        cb = c_ref[b:b + 1, :]
        sc = cb * jax.nn.sigmoid(cb)
        mod = jnp.dot(sc, w1_ref[:, :], preferred_element_type=jnp.float32)
        mod = mod + b1_ref[0:1, :]
        shift = mod[:, :D]
        scale = mod[:, D:]
        cps_x[b].wait()
        xb = x_vmem[b]
        mu = jnp.mean(xb, axis=1, keepdims=True)
        xc = xb - mu
        var = jnp.mean(xc * xc, axis=1, keepdims=True)
        xn = xc * jax.lax.rsqrt(var + 1e-6)
        xm = xn * (1.0 + scale) + shift
        xall[:, D * b:D * (b + 1)] = xm.astype(jnp.bfloat16)

    zall = jnp.dot(xall[:, :], wbig_ref[:, :], preferred_element_type=jnp.float32)
    z0 = zall[:, 0:128]
    z1 = zall[:, 128:256]
    z2 = zall[:, 256:384]

    for i in range(_NCHUNK):
        cps_a[i].wait()
        rows = a_vmem[pl.ds(i * ch, ch), :]
        d_vmem[pl.ds(i * ch, ch), :] = jnp.sum(rows, axis=1, keepdims=True)
        abf[pl.ds(i * ch, ch), :] = rows.astype(jnp.bfloat16)

    s = jax.lax.rsqrt(d_vmem[:, :])
    a = abf[:, :]

    def lap(y):
        u = jnp.dot(a, (s * y).astype(jnp.bfloat16),
                    preferred_element_type=jnp.float32)
        return y - s * u

    t = lap(z2)
    w = lap(z1 + 2.0 * t)
    res = z0 - z2 + w
    OUT = o_ref.shape[2]
    bias = cb_ref[0:1, :]
    for b in range(B):
        o_ref[b] = res[:, OUT * b:OUT * (b + 1)] + bias


def kernel(x, adj, c, W1, b1, cheb_w, cheb_b):
    B, N, D = x.shape
    K, _, _, OUT = cheb_w.shape

    c2 = c.reshape(B, D)
    b1r = b1.reshape(1, 2 * D)
    onehot = (jnp.arange(128)[None, :, None] ==
              (OUT * jnp.arange(B)[:, None, None] +
               jnp.arange(OUT)[None, None, :])).astype(x.dtype)
    wbig = jnp.einsum('kdo,bpo->bdkp', cheb_w[:, 0], onehot)
    wbig = wbig.reshape(B * D, K * 128).astype(jnp.bfloat16)
    cb2 = cheb_b.reshape(1, OUT)

    out = pl.pallas_call(
        _body,
        out_shape=jax.ShapeDtypeStruct((B, N, OUT), jnp.float32),
        in_specs=[
            pl.BlockSpec(memory_space=pl.ANY),
            pl.BlockSpec(memory_space=pl.ANY),
            pl.BlockSpec(memory_space=pltpu.VMEM),
            pl.BlockSpec(memory_space=pltpu.VMEM),
            pl.BlockSpec(memory_space=pltpu.VMEM),
            pl.BlockSpec(memory_space=pltpu.VMEM),
            pl.BlockSpec(memory_space=pltpu.VMEM),
        ],
        scratch_shapes=[
            pltpu.VMEM((N, N), jnp.float32),
            pltpu.VMEM((B, N, D), jnp.float32),
            pltpu.VMEM((N, B * D), jnp.bfloat16),
            pltpu.VMEM((N, N), jnp.bfloat16),
            pltpu.VMEM((N, 1), jnp.float32),
            pltpu.SemaphoreType.DMA((B,)),
            pltpu.SemaphoreType.DMA((_NCHUNK,)),
        ],
        compiler_params=pltpu.CompilerParams(
            vmem_limit_bytes=100 * 1024 * 1024,
        ),
    )(x, adj, c2, W1, b1r, wbig, cb2)

    return out


# speedup vs baseline: 1.1054x; 1.1054x over previous
import jax
import jax.numpy as jnp
from jax.experimental import pallas as pl
from jax.experimental.pallas import tpu as pltpu

def _body(x_hbm, a_hbm, c_ref, w1_ref, b1_ref, wbig_ref, cb_ref, o_ref,
          a_vmem, x_vmem, xall, abf, d_vmem, sem_x, sem_a):
    B, N, D = x_hbm.shape

    cp_x = pltpu.make_async_copy(x_hbm, x_vmem, sem_x)
    cp_a = pltpu.make_async_copy(a_hbm, a_vmem, sem_a)
    cp_x.start()
    cp_a.start()

    cp_x.wait()
    for b in range(B):
        cb = c_ref[b:b + 1, :]
        sc = cb * jax.nn.sigmoid(cb)
        mod = jnp.dot(sc, w1_ref[:, :], preferred_element_type=jnp.float32)
        mod = mod + b1_ref[0:1, :]
        shift = mod[:, :D]
        scale = mod[:, D:]
        xb = x_vmem[b]
        mu = jnp.mean(xb, axis=1, keepdims=True)
        xc = xb - mu
        var = jnp.mean(xc * xc, axis=1, keepdims=True)
        xn = xc * jax.lax.rsqrt(var + 1e-6)
        xm = xn * (1.0 + scale) + shift
        xall[:, D * b:D * (b + 1)] = xm.astype(jnp.bfloat16)

    zall = jnp.dot(xall[:, :], wbig_ref[:, :], preferred_element_type=jnp.float32)
    z0 = zall[:, 0:128]
    z1 = zall[:, 128:256]
    z2 = zall[:, 256:384]

    cp_a.wait()
    rows = a_vmem[:, :]
    d_vmem[:, :] = jnp.sum(rows, axis=1, keepdims=True)
    abf[:, :] = rows.astype(jnp.bfloat16)

    s = jax.lax.rsqrt(d_vmem[:, :])
    a = abf[:, :]

    def lap(y):
        u = jnp.dot(a, (s * y).astype(jnp.bfloat16),
                    preferred_element_type=jnp.float32)
        return y - s * u

    t = lap(z2)
    w = lap(z1 + 2.0 * t)
    res = z0 - z2 + w
    OUT = o_ref.shape[2]
    bias = cb_ref[0:1, :]
    for b in range(B):
        o_ref[b] = res[:, OUT * b:OUT * (b + 1)] + bias


def kernel(x, adj, c, W1, b1, cheb_w, cheb_b):
    B, N, D = x.shape
    K, _, _, OUT = cheb_w.shape

    c2 = c.reshape(B, D)
    b1r = b1.reshape(1, 2 * D)
    onehot = (jnp.arange(128)[None, :, None] ==
              (OUT * jnp.arange(B)[:, None, None] +
               jnp.arange(OUT)[None, None, :])).astype(x.dtype)
    wbig = jnp.einsum('kdo,bpo->bdkp', cheb_w[:, 0], onehot)
    wbig = wbig.reshape(B * D, K * 128).astype(jnp.bfloat16)
    cb2 = cheb_b.reshape(1, OUT)

    out = pl.pallas_call(
        _body,
        out_shape=jax.ShapeDtypeStruct((B, N, OUT), jnp.float32),
        in_specs=[
            pl.BlockSpec(memory_space=pl.ANY),
            pl.BlockSpec(memory_space=pl.ANY),
            pl.BlockSpec(memory_space=pltpu.VMEM),
            pl.BlockSpec(memory_space=pltpu.VMEM),
            pl.BlockSpec(memory_space=pltpu.VMEM),
            pl.BlockSpec(memory_space=pltpu.VMEM),
            pl.BlockSpec(memory_space=pltpu.VMEM),
        ],
        scratch_shapes=[
            pltpu.VMEM((N, N), jnp.float32),
            pltpu.VMEM((B, N, D), jnp.float32),
            pltpu.VMEM((N, B * D), jnp.bfloat16),
            pltpu.VMEM((N, N), jnp.bfloat16),
            pltpu.VMEM((N, 1), jnp.float32),
            pltpu.SemaphoreType.DMA,
            pltpu.SemaphoreType.DMA,
        ],
        compiler_params=pltpu.CompilerParams(
            vmem_limit_bytes=100 * 1024 * 1024,
        ),
    )(x, adj, c2, W1, b1r, wbig, cb2)

    return out
